# baseline (speedup 1.0000x reference)
import dataclasses
import functools

import jax
import jax.numpy as jnp
from jax import lax
from jax.experimental import pallas as pl
from jax.experimental.pallas import tpu as pltpu
from jax.experimental.pallas import tpu_sc as plsc

N = 10000
E = 320000
D_IN = 128
H = 150
HP = 160
HC = 80
HCB = 96
L = 3
G3 = 3 * HP

NC = 2
NS = 16
K = 80
EPAD = E
EPS = EPAD // NS
NCHUNK = EPS // K
NBUF = 5
GLA = 3
NP = 10240
ZR = 128
RPS = NP // NS


def _split_halves(res, o_ref):
    z = jnp.zeros((res.shape[0], HCB - HC), jnp.float32)
    o_ref[0] = jnp.concatenate([res[:, :HC], z], axis=1).astype(jnp.bfloat16)
    o_ref[1] = jnp.concatenate([res[:, HC:], z], axis=1).astype(jnp.bfloat16)


def _mm_body(x_ref, w_ref, o_ref):
    res = jnp.dot(x_ref[...], w_ref[...], preferred_element_type=jnp.float32)
    _split_halves(res, o_ref)


def _tc_matmul(x, w, r_blk=2000):
    return pl.pallas_call(
        _mm_body,
        grid=(N // r_blk,),
        in_specs=[pl.BlockSpec((r_blk, HP), lambda i: (i, 0)),
                  pl.BlockSpec((HP, HP), lambda i: (0, 0))],
        out_specs=pl.BlockSpec((NC, r_blk, HCB), lambda i: (0, i, 0)),
        out_shape=jax.ShapeDtypeStruct((NC, N, HCB), jnp.bfloat16),
    )(x, w)


def _gh_body(x_ref, whh_ref, bhh_ref, o_ref):
    o_ref[...] = jnp.dot(x_ref[...], whh_ref[...],
                         preferred_element_type=jnp.float32) + bhh_ref[...]


def _tc_gh(x, whh, bhh, r_blk=2000):
    return pl.pallas_call(
        _gh_body,
        grid=(N // r_blk,),
        in_specs=[pl.BlockSpec((r_blk, HP), lambda i: (i, 0)),
                  pl.BlockSpec((HP, G3), lambda i: (0, 0)),
                  pl.BlockSpec((1, G3), lambda i: (0, 0))],
        out_specs=pl.BlockSpec((r_blk, G3), lambda i: (i, 0)),
        out_shape=jax.ShapeDtypeStruct((N, G3), jnp.float32),
    )(x, whh, bhh)


def _gru_math(p_ref, x_ref, gh_ref, wih_ref, bih_ref):
    agg = jnp.concatenate([p_ref[0], p_ref[1]], axis=-1)
    x = x_ref[...]
    gi = jnp.dot(agg, wih_ref[...],
                 preferred_element_type=jnp.float32) + bih_ref[...]
    gh = gh_ref[...]
    r = jax.nn.sigmoid(gi[:, :HP] + gh[:, :HP])
    z = jax.nn.sigmoid(gi[:, HP:2 * HP] + gh[:, HP:2 * HP])
    n = jnp.tanh(gi[:, 2 * HP:] + r * gh[:, 2 * HP:])
    return (1.0 - z) * n + z * x


def _post_body(p_ref, x_ref, gh_ref, wih_ref, bih_ref, wn_ref,
               xo_ref, mo_ref):
    xn = _gru_math(p_ref, x_ref, gh_ref, wih_ref, bih_ref)
    xo_ref[...] = xn
    res = jnp.dot(xn, wn_ref[...], preferred_element_type=jnp.float32)
    _split_halves(res, mo_ref)


def _post_final_body(p_ref, x_ref, gh_ref, wih_ref, bih_ref, xo_ref):
    xo_ref[...] = _gru_math(p_ref, x_ref, gh_ref, wih_ref, bih_ref)


_POST_SPECS = [pl.BlockSpec((NC, 2000, HCB), lambda i: (0, i, 0)),
               pl.BlockSpec((2000, HP), lambda i: (i, 0)),
               pl.BlockSpec((2000, G3), lambda i: (i, 0)),
               pl.BlockSpec((NC * HCB, G3), lambda i: (0, 0)),
               pl.BlockSpec((1, G3), lambda i: (0, 0))]


def _tc_post(p, x, gh, wih, bih, wnext):
    return pl.pallas_call(
        _post_body,
        grid=(N // 2000,),
        in_specs=_POST_SPECS + [pl.BlockSpec((HP, HP), lambda i: (0, 0))],
        out_specs=[pl.BlockSpec((2000, HP), lambda i: (i, 0)),
                   pl.BlockSpec((NC, 2000, HCB), lambda i: (0, i, 0))],
        out_shape=[jax.ShapeDtypeStruct((N, HP), jnp.float32),
                   jax.ShapeDtypeStruct((NC, N, HCB), jnp.bfloat16)],
    )(p, x, gh, wih, bih, wnext)


def _tc_post_final(p, x, gh, wih, bih):
    return pl.pallas_call(
        _post_final_body,
        grid=(N // 2000,),
        in_specs=_POST_SPECS,
        out_specs=pl.BlockSpec((2000, HP), lambda i: (i, 0)),
        out_shape=jax.ShapeDtypeStruct((N, HP), jnp.float32),
    )(p, x, gh, wih, bih)


def _edge_body(m_hbm, epk_hbm, out_hbm, *scr):
    pks = list(scr[0:NBUF])
    rowss = list(scr[NBUF:2 * NBUF])
    fbuf = scr[2 * NBUF]
    zero_v = scr[2 * NBUF + 1]
    acc_sp = scr[2 * NBUF + 2]
    isems = list(scr[2 * NBUF + 3:3 * NBUF + 3])
    gsems = list(scr[3 * NBUF + 3:4 * NBUF + 3])
    cid = lax.axis_index("c")
    sid = lax.axis_index("s")

    zvec = jnp.zeros((16,), jnp.float32)

    @pl.loop(0, ZR)
    def _(r):
        for j in range(HCB // 16):
            zero_v[r, pl.ds(j * 16, 16)] = zvec

    @pl.loop(0, RPS // ZR)
    def _(t):
        pltpu.sync_copy(zero_v, acc_sp.at[pl.ds(sid * RPS + t * ZR, ZR)])

    plsc.subcore_barrier()

    cbase = sid * NCHUNK

    def idx_start(ck, pk, sem):
        pltpu.make_async_copy(epk_hbm.at[cbase + ck], pk, sem).start()

    def idx_wait(pk, sem):
        pltpu.make_async_copy(epk_hbm.at[cbase], pk, sem).wait()

    def gstart(pk, rows, sem):
        pltpu.make_async_copy(m_hbm.at[cid].at[pk.at[0]], rows, sem).start()

    def gwait(pk, rows, sem):
        pltpu.make_async_copy(m_hbm.at[cid].at[pk.at[0]], rows, sem).wait()

    def process(pk, rows):
        @plsc.parallel_loop(0, K, unroll=8)
        def _(e):
            wi = plsc.load_gather(pk, [jnp.full((16,), 2, jnp.int32),
                                       jnp.full((16,), e, jnp.int32)])
            ws = plsc.bitcast(wi, jnp.float32)
            wsb = plsc.pack(ws, ws, format=plsc.PackFormat.INTERLEAVED)
            for j in range(HCB // 32):
                v = rows[e, pl.ds(j * 32, 32)] * wsb
                lo, hi = plsc.unpack(v, format=plsc.PackFormat.INTERLEAVED)
                fbuf[e, pl.ds(j * 32, 16)] = lo
                fbuf[e, pl.ds(j * 32 + 16, 16)] = hi

        pltpu.sync_copy(fbuf, acc_sp.at[pk.at[1]], add=True)

    for b in range(NBUF):
        idx_start(b, pks[b], isems[b])
    for b in range(GLA):
        idx_wait(pks[b], isems[b])
        gstart(pks[b], rowss[b], gsems[b])

    @pl.loop(0, NCHUNK // NBUF)
    def _(g):
        c0 = g * NBUF
        for b in range(NBUF):
            c = c0 + b
            gwait(pks[b], rowss[b], gsems[b])
            process(pks[b], rowss[b])

            @pl.when(c + NBUF < NCHUNK)
            def _():
                idx_start(c + NBUF, pks[b], isems[b])

            b3 = (b + GLA) % NBUF

            @pl.when(c + GLA < NCHUNK)
            def _():
                idx_wait(pks[b3], isems[b3])
                gstart(pks[b3], rowss[b3], gsems[b3])

    plsc.subcore_barrier()

    @pl.loop(0, RPS // ZR)
    def _(t):
        r0 = sid * RPS + t * ZR
        pltpu.sync_copy(acc_sp.at[pl.ds(r0, ZR)],
                        out_hbm.at[cid].at[pl.ds(r0, ZR)])


def _sc_compiler_params():
    cp = pltpu.CompilerParams()
    if "needs_layout_passes" in pltpu.CompilerParams.__dataclass_fields__:
        cp = dataclasses.replace(cp, needs_layout_passes=False)
    if "use_tc_tiling_on_sc" in pltpu.CompilerParams.__dataclass_fields__:
        cp = dataclasses.replace(cp, use_tc_tiling_on_sc=False)
    return cp


def _sc_edge(m, epk):
    mesh = plsc.VectorSubcoreMesh(core_axis_name="c", subcore_axis_name="s")
    f = pl.kernel(
        _edge_body,
        out_type=jax.ShapeDtypeStruct((NC, NP, HCB), jnp.float32),
        mesh=mesh,
        compiler_params=_sc_compiler_params(),
        scratch_types=(
            [pltpu.VMEM((3, K), jnp.int32) for _ in range(NBUF)]
            + [pltpu.VMEM((K, HCB), jnp.bfloat16) for _ in range(NBUF)]
            + [pltpu.VMEM((K, HCB), jnp.float32),
               pltpu.VMEM((ZR, HCB), jnp.float32),
               pltpu.VMEM_SHARED((NP, HCB), jnp.float32)]
            + [pltpu.SemaphoreType.DMA for _ in range(2 * NBUF)]
        ),
    )
    return f(m, epk)


def _pack_edges(src, dst, w):
    return jnp.stack([
        src.reshape(EPAD // K, K),
        dst.reshape(EPAD // K, K),
        lax.bitcast_convert_type(w, jnp.int32).reshape(EPAD // K, K),
    ], axis=1)


def _prep_conv_w(weight):
    return jnp.pad(weight, ((0, 0), (0, HP - H), (0, HP - H)))


def _prep_gru_w(w):
    blocks = [jnp.pad(w[g * H:(g + 1) * H, :].T,
                      ((0, HP - H), (0, HP - H))) for g in range(3)]
    return jnp.concatenate(blocks, axis=1)


def _expand_gru_w(wp):
    rows = []
    for c in range(NC):
        for q in range(HCB):
            j, t = divmod(q, 32)
            lcol = 32 * j + (2 * t if t < 16 else 2 * (t - 16) + 1)
            if lcol < HC:
                rows.append(wp[c * HC + lcol])
            else:
                rows.append(jnp.zeros((G3,), jnp.float32))
    return jnp.stack(rows)


def _prep_gru_b(b):
    bs = [jnp.pad(b[g * H:(g + 1) * H], (0, HP - H)) for g in range(3)]
    return jnp.concatenate(bs)[None, :]


def _ggc(xp, epk, conv_w, wih, whh, bih, bhh, next_w0):
    wih_x = _expand_gru_w(wih)
    m = _tc_matmul(xp, conv_w[0])
    for i in range(L):
        gh = _tc_gh(xp, whh, bhh)
        p = _sc_edge(m, epk)
        if i + 1 < L:
            xp, m = _tc_post(p, xp, gh, wih_x, bih, conv_w[i + 1])
        elif next_w0 is not None:
            xp, m = _tc_post(p, xp, gh, wih_x, bih, next_w0)
        else:
            xp = _tc_post_final(p, xp, gh, wih_x, bih)
    return xp


def kernel(x, edge_index, edge_attr, enc_weight, enc_w_ih, enc_w_hh,
           enc_b_ih, enc_b_hh, dec_weight, dec_w_ih, dec_w_hh,
           dec_b_ih, dec_b_hh):
    src = edge_index[0].astype(jnp.int32)
    dst = edge_index[1].astype(jnp.int32)
    w = edge_attr.astype(jnp.float32)
    epk = _pack_edges(src, dst, w)

    xp = jnp.pad(x, ((0, 0), (0, HP - D_IN)))

    h = _ggc(xp, epk, _prep_conv_w(enc_weight),
             _prep_gru_w(enc_w_ih), _prep_gru_w(enc_w_hh),
             _prep_gru_b(enc_b_ih), _prep_gru_b(enc_b_hh), None)
    r = _ggc(h, epk, _prep_conv_w(dec_weight),
             _prep_gru_w(dec_w_ih), _prep_gru_w(dec_w_hh),
             _prep_gru_b(dec_b_ih), _prep_gru_b(dec_b_hh), None)
    return h[:, :H], r[:, :H]


# speedup vs baseline: 1.0746x; 1.0746x over previous
import dataclasses
import functools

import jax
import jax.numpy as jnp
from jax import lax
from jax.experimental import pallas as pl
from jax.experimental.pallas import tpu as pltpu
from jax.experimental.pallas import tpu_sc as plsc

N = 10000
E = 320000
D_IN = 128
H = 150
HP = 160
HC = 80
L = 3
G3 = 3 * HP

NC = 2
NS = 16
K = 80
EPAD = E
EPS = EPAD // NS
NCHUNK = EPS // K
NBUF = 5
GLA = 3
NP = 10240
ZR = 128
RPS = NP // NS


def _split_halves(res, o_ref):
    o_ref[0] = res[:, :HC]
    o_ref[1] = res[:, HC:]


def _mm_body(x_ref, w_ref, o_ref):
    res = jnp.dot(x_ref[...], w_ref[...], preferred_element_type=jnp.float32)
    _split_halves(res, o_ref)


def _tc_matmul(x, w, r_blk=2000):
    return pl.pallas_call(
        _mm_body,
        grid=(N // r_blk,),
        in_specs=[pl.BlockSpec((r_blk, HP), lambda i: (i, 0)),
                  pl.BlockSpec((HP, HP), lambda i: (0, 0))],
        out_specs=pl.BlockSpec((NC, r_blk, HC), lambda i: (0, i, 0)),
        out_shape=jax.ShapeDtypeStruct((NC, N, HC), jnp.float32),
    )(x, w)


def _gh_body(x_ref, whh_ref, bhh_ref, o_ref):
    o_ref[...] = jnp.dot(x_ref[...], whh_ref[...],
                         preferred_element_type=jnp.float32) + bhh_ref[...]


def _tc_gh(x, whh, bhh, r_blk=2000):
    return pl.pallas_call(
        _gh_body,
        grid=(N // r_blk,),
        in_specs=[pl.BlockSpec((r_blk, HP), lambda i: (i, 0)),
                  pl.BlockSpec((HP, G3), lambda i: (0, 0)),
                  pl.BlockSpec((1, G3), lambda i: (0, 0))],
        out_specs=pl.BlockSpec((r_blk, G3), lambda i: (i, 0)),
        out_shape=jax.ShapeDtypeStruct((N, G3), jnp.float32),
    )(x, whh, bhh)


def _gru_math(p_ref, x_ref, gh_ref, wih_ref, bih_ref):
    agg = jnp.concatenate([p_ref[0], p_ref[1]], axis=-1)
    x = x_ref[...]
    gi = jnp.dot(agg, wih_ref[...],
                 preferred_element_type=jnp.float32) + bih_ref[...]
    gh = gh_ref[...]
    r = jax.nn.sigmoid(gi[:, :HP] + gh[:, :HP])
    z = jax.nn.sigmoid(gi[:, HP:2 * HP] + gh[:, HP:2 * HP])
    n = jnp.tanh(gi[:, 2 * HP:] + r * gh[:, 2 * HP:])
    return (1.0 - z) * n + z * x


def _post_body(p_ref, x_ref, gh_ref, wih_ref, bih_ref, wn_ref,
               xo_ref, mo_ref):
    xn = _gru_math(p_ref, x_ref, gh_ref, wih_ref, bih_ref)
    xo_ref[...] = xn
    res = jnp.dot(xn, wn_ref[...], preferred_element_type=jnp.float32)
    _split_halves(res, mo_ref)


def _post_final_body(p_ref, x_ref, gh_ref, wih_ref, bih_ref, xo_ref):
    xo_ref[...] = _gru_math(p_ref, x_ref, gh_ref, wih_ref, bih_ref)


_POST_SPECS = [pl.BlockSpec((NC, 2000, HC), lambda i: (0, i, 0)),
               pl.BlockSpec((2000, HP), lambda i: (i, 0)),
               pl.BlockSpec((2000, G3), lambda i: (i, 0)),
               pl.BlockSpec((HP, G3), lambda i: (0, 0)),
               pl.BlockSpec((1, G3), lambda i: (0, 0))]


def _tc_post(p, x, gh, wih, bih, wnext):
    return pl.pallas_call(
        _post_body,
        grid=(N // 2000,),
        in_specs=_POST_SPECS + [pl.BlockSpec((HP, HP), lambda i: (0, 0))],
        out_specs=[pl.BlockSpec((2000, HP), lambda i: (i, 0)),
                   pl.BlockSpec((NC, 2000, HC), lambda i: (0, i, 0))],
        out_shape=[jax.ShapeDtypeStruct((N, HP), jnp.float32),
                   jax.ShapeDtypeStruct((NC, N, HC), jnp.float32)],
    )(p, x, gh, wih, bih, wnext)


def _tc_post_final(p, x, gh, wih, bih):
    return pl.pallas_call(
        _post_final_body,
        grid=(N // 2000,),
        in_specs=_POST_SPECS,
        out_specs=pl.BlockSpec((2000, HP), lambda i: (i, 0)),
        out_shape=jax.ShapeDtypeStruct((N, HP), jnp.float32),
    )(p, x, gh, wih, bih)


def _edge_body(m_hbm, epk_hbm, out_hbm, *scr):
    pks = list(scr[0:NBUF])
    rowss = list(scr[NBUF:2 * NBUF])
    zero_v = scr[2 * NBUF]
    acc_sp = scr[2 * NBUF + 1]
    isems = list(scr[2 * NBUF + 2:3 * NBUF + 2])
    gsems = list(scr[3 * NBUF + 2:4 * NBUF + 2])
    cid = lax.axis_index("c")
    sid = lax.axis_index("s")

    zvec = jnp.zeros((16,), jnp.float32)

    @pl.loop(0, ZR)
    def _(r):
        for j in range(HC // 16):
            zero_v[r, pl.ds(j * 16, 16)] = zvec

    @pl.loop(0, RPS // ZR)
    def _(t):
        pltpu.sync_copy(zero_v, acc_sp.at[pl.ds(sid * RPS + t * ZR, ZR)])

    plsc.subcore_barrier()

    cbase = sid * NCHUNK

    def idx_start(ck, pk, sem):
        pltpu.make_async_copy(epk_hbm.at[cbase + ck], pk, sem).start()

    def idx_wait(pk, sem):
        pltpu.make_async_copy(epk_hbm.at[cbase], pk, sem).wait()

    def gstart(pk, rows, sem):
        pltpu.make_async_copy(m_hbm.at[cid].at[pk.at[0]], rows, sem).start()

    def gwait(pk, rows, sem):
        pltpu.make_async_copy(m_hbm.at[cid].at[pk.at[0]], rows, sem).wait()

    def process(pk, rows):
        @plsc.parallel_loop(0, K, unroll=8)
        def _(e):
            wi = plsc.load_gather(pk, [jnp.full((16,), 2, jnp.int32),
                                       jnp.full((16,), e, jnp.int32)])
            ws = plsc.bitcast(wi, jnp.float32)
            for j in range(HC // 16):
                slc = pl.ds(j * 16, 16)
                rows[e, slc] = rows[e, slc] * ws

        pltpu.sync_copy(rows, acc_sp.at[pk.at[1]], add=True)

    for b in range(NBUF):
        idx_start(b, pks[b], isems[b])
    for b in range(GLA):
        idx_wait(pks[b], isems[b])
        gstart(pks[b], rowss[b], gsems[b])

    @pl.loop(0, NCHUNK // NBUF)
    def _(g):
        c0 = g * NBUF
        for b in range(NBUF):
            c = c0 + b
            gwait(pks[b], rowss[b], gsems[b])
            process(pks[b], rowss[b])

            @pl.when(c + NBUF < NCHUNK)
            def _():
                idx_start(c + NBUF, pks[b], isems[b])

            b3 = (b + GLA) % NBUF

            @pl.when(c + GLA < NCHUNK)
            def _():
                idx_wait(pks[b3], isems[b3])
                gstart(pks[b3], rowss[b3], gsems[b3])

    plsc.subcore_barrier()

    @pl.loop(0, RPS // ZR)
    def _(t):
        r0 = sid * RPS + t * ZR
        pltpu.sync_copy(acc_sp.at[pl.ds(r0, ZR)],
                        out_hbm.at[cid].at[pl.ds(r0, ZR)])


def _sc_compiler_params():
    cp = pltpu.CompilerParams()
    if "needs_layout_passes" in pltpu.CompilerParams.__dataclass_fields__:
        cp = dataclasses.replace(cp, needs_layout_passes=False)
    if "use_tc_tiling_on_sc" in pltpu.CompilerParams.__dataclass_fields__:
        cp = dataclasses.replace(cp, use_tc_tiling_on_sc=False)
    return cp


def _sc_edge(m, epk):
    mesh = plsc.VectorSubcoreMesh(core_axis_name="c", subcore_axis_name="s")
    f = pl.kernel(
        _edge_body,
        out_type=jax.ShapeDtypeStruct((NC, NP, HC), jnp.float32),
        mesh=mesh,
        compiler_params=_sc_compiler_params(),
        scratch_types=(
            [pltpu.VMEM((3, K), jnp.int32) for _ in range(NBUF)]
            + [pltpu.VMEM((K, HC), jnp.float32) for _ in range(NBUF)]
            + [pltpu.VMEM((ZR, HC), jnp.float32),
               pltpu.VMEM_SHARED((NP, HC), jnp.float32)]
            + [pltpu.SemaphoreType.DMA for _ in range(2 * NBUF)]
        ),
    )
    return f(m, epk)


def _pack_edges(src, dst, w):
    return jnp.stack([
        src.reshape(EPAD // K, K),
        dst.reshape(EPAD // K, K),
        lax.bitcast_convert_type(w, jnp.int32).reshape(EPAD // K, K),
    ], axis=1)


def _prep_conv_w(weight):
    return jnp.pad(weight, ((0, 0), (0, HP - H), (0, HP - H)))


def _prep_gru_w(w):
    blocks = [jnp.pad(w[g * H:(g + 1) * H, :].T,
                      ((0, HP - H), (0, HP - H))) for g in range(3)]
    return jnp.concatenate(blocks, axis=1)


def _prep_gru_b(b):
    bs = [jnp.pad(b[g * H:(g + 1) * H], (0, HP - H)) for g in range(3)]
    return jnp.concatenate(bs)[None, :]


def _ggc(xp, epk, conv_w, wih, whh, bih, bhh, next_w0):
    m = _tc_matmul(xp, conv_w[0])
    for i in range(L):
        gh = _tc_gh(xp, whh, bhh)
        p = _sc_edge(m, epk)
        if i + 1 < L:
            xp, m = _tc_post(p, xp, gh, wih, bih, conv_w[i + 1])
        elif next_w0 is not None:
            xp, m = _tc_post(p, xp, gh, wih, bih, next_w0)
        else:
            xp = _tc_post_final(p, xp, gh, wih, bih)
    return xp


def kernel(x, edge_index, edge_attr, enc_weight, enc_w_ih, enc_w_hh,
           enc_b_ih, enc_b_hh, dec_weight, dec_w_ih, dec_w_hh,
           dec_b_ih, dec_b_hh):
    src = edge_index[0].astype(jnp.int32)
    dst = edge_index[1].astype(jnp.int32)
    w = edge_attr.astype(jnp.float32)
    epk = _pack_edges(src, dst, w)

    xp = jnp.pad(x, ((0, 0), (0, HP - D_IN)))

    h = _ggc(xp, epk, _prep_conv_w(enc_weight),
             _prep_gru_w(enc_w_ih), _prep_gru_w(enc_w_hh),
             _prep_gru_b(enc_b_ih), _prep_gru_b(enc_b_hh), None)
    r = _ggc(h, epk, _prep_conv_w(dec_weight),
             _prep_gru_w(dec_w_ih), _prep_gru_w(dec_w_hh),
             _prep_gru_b(dec_b_ih), _prep_gru_b(dec_b_hh), None)
    return h[:, :H], r[:, :H]
